# Initial kernel scaffold; baseline (speedup 1.0000x reference)
import jax
import jax.numpy as jnp
from jax.experimental import pallas as pl


def kernel(x, edge_index, W1, b1, W2, b2):
    raise NotImplementedError("write your pallas kernel here")


# speedup vs baseline: 10.7144x; 10.7144x over previous
import functools

import jax
import jax.numpy as jnp
from jax import lax
from jax.experimental import pallas as pl
from jax.experimental.pallas import tpu as pltpu
from jax.experimental.pallas import tpu_sc as plsc

NC = 2
NS = 16
NW = NC * NS
EB = 128


def _sc_degree(dstp, zed, ones, n_pad):
    e_pad = dstp.shape[0]
    epw = e_pad // NW
    nb = epw // EB
    rpt = n_pad // NS
    mesh = plsc.VectorSubcoreMesh(core_axis_name="c", subcore_axis_name="s", num_cores=NC, num_subcores=NS)

    @functools.partial(
        pl.kernel,
        out_type=jax.ShapeDtypeStruct((NC, n_pad, 16), jnp.float32),
        mesh=mesh,
        scratch_types=[
            pltpu.VMEM((EB,), jnp.int32),
            pltpu.VMEM((EB, 16), jnp.float32),
            pltpu.VMEM_SHARED((n_pad, 16), jnp.float32),
        ],
    )
    def k(dst_hbm, zed_hbm, ones_hbm, out_hbm, idx_v, ones_v, deg_sh):
        c = lax.axis_index("c")
        s = lax.axis_index("s")
        wid = s * NC + c
        pltpu.sync_copy(ones_hbm, ones_v)
        pltpu.sync_copy(zed_hbm.at[pl.ds(s * rpt, rpt)],
                        deg_sh.at[pl.ds(s * rpt, rpt)])
        plsc.subcore_barrier()

        def step(b, carry):
            base = wid * epw + b * EB
            pltpu.sync_copy(dst_hbm.at[pl.ds(base, EB)], idx_v)
            pltpu.sync_copy(ones_v, deg_sh.at[idx_v], add=True)
            return carry

        lax.fori_loop(0, nb, step, 0)
        plsc.subcore_barrier()
        pltpu.sync_copy(deg_sh.at[pl.ds(s * rpt, rpt)],
                        out_hbm.at[c, pl.ds(s * rpt, rpt)])

    return k(dstp, zed, ones)


def _sc_spmm(g, srcp, dstp, zacc, n_pad):
    d = g.shape[1]
    e_pad = srcp.shape[0]
    epw = e_pad // NW
    nb = epw // EB
    rpt = n_pad // NS
    mesh = plsc.VectorSubcoreMesh(core_axis_name="c", subcore_axis_name="s", num_cores=NC, num_subcores=NS)

    @functools.partial(
        pl.kernel,
        out_type=jax.ShapeDtypeStruct((NC, n_pad, d), jnp.float32),
        mesh=mesh,
        scratch_types=[
            pltpu.VMEM((EB,), jnp.int32),
            pltpu.VMEM((EB,), jnp.int32),
            pltpu.VMEM((EB, d), jnp.float32),
            pltpu.SemaphoreType.DMA,
            pltpu.VMEM_SHARED((n_pad, d), jnp.float32),
        ],
    )
    def k(g_hbm, src_hbm, dst_hbm, zacc_hbm, out_hbm,
          idx_s, idx_d, rows_v, sem, acc_sh):
        c = lax.axis_index("c")
        s = lax.axis_index("s")
        wid = s * NC + c
        pltpu.sync_copy(zacc_hbm.at[pl.ds(s * rpt, rpt)],
                        acc_sh.at[pl.ds(s * rpt, rpt)])
        plsc.subcore_barrier()

        def step(b, carry):
            base = wid * epw + b * EB
            pltpu.sync_copy(src_hbm.at[pl.ds(base, EB)], idx_s)
            pltpu.sync_copy(dst_hbm.at[pl.ds(base, EB)], idx_d)
            pltpu.async_copy(g_hbm.at[idx_s], rows_v, sem).wait()
            pltpu.sync_copy(rows_v, acc_sh.at[idx_d], add=True)
            return carry

        lax.fori_loop(0, nb, step, 0)
        plsc.subcore_barrier()
        pltpu.sync_copy(acc_sh.at[pl.ds(s * rpt, rpt)],
                        out_hbm.at[c, pl.ds(s * rpt, rpt)])

    return k(g, srcp, dstp, zacc)


def _dis_block(dg_ref):
    return lax.rsqrt(dg_ref[0, :, 0:1] + dg_ref[1, :, 0:1] + 1.0)


def _tc_scale_matmul(x, w, deg16, bm):
    n, d = x.shape

    def body(x_ref, w_ref, dg_ref, out_ref):
        dis = _dis_block(dg_ref)
        out_ref[...] = dis * jnp.dot(x_ref[...], w_ref[...],
                                     preferred_element_type=jnp.float32)

    return pl.pallas_call(
        body,
        grid=(n // bm,),
        in_specs=[
            pl.BlockSpec((bm, d), lambda i: (i, 0)),
            pl.BlockSpec((d, d), lambda i: (0, 0)),
            pl.BlockSpec((2, bm, 16), lambda i: (0, i, 0)),
        ],
        out_specs=pl.BlockSpec((bm, d), lambda i: (i, 0)),
        out_shape=jax.ShapeDtypeStruct((n, d), jnp.float32),
    )(x, w, deg16)


def _tc_mid(acc, g1, deg16, b1, w2, bm):
    n, d = g1.shape

    def body(acc_ref, g1_ref, dg_ref, b1_ref, w2_ref, out_ref):
        dis = _dis_block(dg_ref)
        o1 = dis * (acc_ref[0] + acc_ref[1] + g1_ref[...]) + b1_ref[...]
        o1 = jnp.maximum(o1, 0.0)
        out_ref[...] = dis * jnp.dot(o1, w2_ref[...],
                                     preferred_element_type=jnp.float32)

    return pl.pallas_call(
        body,
        grid=(n // bm,),
        in_specs=[
            pl.BlockSpec((2, bm, d), lambda i: (0, i, 0)),
            pl.BlockSpec((bm, d), lambda i: (i, 0)),
            pl.BlockSpec((2, bm, 16), lambda i: (0, i, 0)),
            pl.BlockSpec((1, d), lambda i: (0, 0)),
            pl.BlockSpec((d, d), lambda i: (0, 0)),
        ],
        out_specs=pl.BlockSpec((bm, d), lambda i: (i, 0)),
        out_shape=jax.ShapeDtypeStruct((n, d), jnp.float32),
    )(acc, g1, deg16, b1, w2)


def _tc_final(acc, g2, deg16, b2, bm):
    n, d = g2.shape

    def body(acc_ref, g2_ref, dg_ref, b2_ref, out_ref):
        dis = _dis_block(dg_ref)
        out_ref[...] = dis * (acc_ref[0] + acc_ref[1] + g2_ref[...]) + b2_ref[...]

    return pl.pallas_call(
        body,
        grid=(n // bm,),
        in_specs=[
            pl.BlockSpec((2, bm, d), lambda i: (0, i, 0)),
            pl.BlockSpec((bm, d), lambda i: (i, 0)),
            pl.BlockSpec((2, bm, 16), lambda i: (0, i, 0)),
            pl.BlockSpec((1, d), lambda i: (0, 0)),
        ],
        out_specs=pl.BlockSpec((bm, d), lambda i: (i, 0)),
        out_shape=jax.ShapeDtypeStruct((n, d), jnp.float32),
    )(acc, g2, deg16, b2)


def kernel(x, edge_index, W1, b1, W2, b2):
    n, d = x.shape
    e = edge_index.shape[1]
    src = edge_index[0].astype(jnp.int32)
    dst = edge_index[1].astype(jnp.int32)

    chunk = NW * EB
    e_pad = -(-e // chunk) * chunk
    pad = e_pad - e
    srcp = jnp.concatenate([src, jnp.zeros((pad,), jnp.int32)])
    dstp = jnp.concatenate([dst, jnp.full((pad,), n, jnp.int32)])

    n_pad = -(-(n + 1) // (NS * 8)) * (NS * 8)
    zed = jnp.zeros((n_pad, 16), jnp.float32)
    ones = jnp.ones((EB, 16), jnp.float32)
    zacc = jnp.zeros((n_pad, d), jnp.float32)

    deg16 = _sc_degree(dstp, zed, ones, n_pad)[:, :n]

    bm = 1000 if n % 1000 == 0 else 8
    b1r = b1.reshape(1, d)
    b2r = b2.reshape(1, d)

    g1 = _tc_scale_matmul(x, W1, deg16, bm)
    acc1 = _sc_spmm(g1, srcp, dstp, zacc, n_pad)[:, :n]
    g2 = _tc_mid(acc1, g1, deg16, b1r, W2, bm)
    acc2 = _sc_spmm(g2, srcp, dstp, zacc, n_pad)[:, :n]
    return _tc_final(acc2, g2, deg16, b2r, bm)
